# Initial kernel scaffold; baseline (speedup 1.0000x reference)
import jax
import jax.numpy as jnp
from jax.experimental import pallas as pl


def kernel(x1, pos1, edge_attr1, edge_index1, triple_index1, quadra_index1, batch1, x2, pos2, edge_attr2, edge_index2, triple_index2, quadra_index2, batch2, molar_ratio, temps, params):
    raise NotImplementedError("write your pallas kernel here")


# speedup vs baseline: 5.6466x; 5.6466x over previous
import functools

import jax
import jax.numpy as jnp
from jax import lax
from jax.experimental import pallas as pl
from jax.experimental.pallas import tpu as pltpu
from jax.experimental.pallas import tpu_sc as plsc

N_NODES = 10000
N_EDGES = 320000
NB = 256
HID = 64
ENC_DIM = 128
ENC_HEADS = 8
ENC_LAYERS = 3
FF = 512
N_ENERGY = 64

NC = 2
NS = 16
NW = NC * NS
EPT = N_EDGES // NW
CH = 80
NCH = EPT // CH
NPAD = 10240
NPT = NPAD // NS

_mesh = plsc.VectorSubcoreMesh(core_axis_name="c", subcore_axis_name="s")
_sc_params = pltpu.CompilerParams(needs_layout_passes=False, use_tc_tiling_on_sc=False)


def _rsqrt16(x):
    i = plsc.bitcast(x, jnp.int32)
    i = jnp.int32(0x5F3759DF) - (i >> 1)
    y = plsc.bitcast(i, jnp.float32)
    y = y * (1.5 - 0.5 * x * y * y)
    y = y * (1.5 - 0.5 * x * y * y)
    y = y * (1.5 - 0.5 * x * y * y)
    return y


def _geom_body(px_h, py_h, pz_h, se_h, de_h, ti_h, tj_h, tk_h, qa_h, qb_h,
               qc_h, qd_h, dist_h, ct_h, cq_h, px, py, pz, i0, i1, i2, i3,
               ov):
    c = lax.axis_index("c")
    s = lax.axis_index("s")
    w = c * NS + s
    sl_w = pl.ds(w * EPT, EPT)
    pltpu.sync_copy(px_h, px)
    pltpu.sync_copy(py_h, py)
    pltpu.sync_copy(pz_h, pz)

    def gxyz(idx):
        return (plsc.load_gather(px, [idx]),
                plsc.load_gather(py, [idx]),
                plsc.load_gather(pz, [idx]))

    pltpu.sync_copy(se_h.at[sl_w], i0)
    pltpu.sync_copy(de_h.at[sl_w], i1)

    def ebody(i, carry):
        a16 = i0[pl.ds(i * 16, 16)]
        b16 = i1[pl.ds(i * 16, 16)]
        ax, ay, az = gxyz(a16)
        bx, by, bz = gxyz(b16)
        ux, uy, uz = ax - bx, ay - by, az - bz
        n2 = ux * ux + uy * uy + uz * uz + 1e-12
        ov[pl.ds(i * 16, 16)] = n2 * _rsqrt16(n2)
        return carry

    lax.fori_loop(0, EPT // 16, ebody, 0)
    pltpu.sync_copy(ov, dist_h.at[sl_w])

    pltpu.sync_copy(ti_h.at[sl_w], i0)
    pltpu.sync_copy(tj_h.at[sl_w], i1)
    pltpu.sync_copy(tk_h.at[sl_w], i2)

    def tbody(i, carry):
        a16 = i0[pl.ds(i * 16, 16)]
        b16 = i1[pl.ds(i * 16, 16)]
        c16 = i2[pl.ds(i * 16, 16)]
        ix, iy, iz = gxyz(a16)
        jx, jy, jz = gxyz(b16)
        kx, ky, kz = gxyz(c16)
        v1x, v1y, v1z = ix - jx, iy - jy, iz - jz
        v2x, v2y, v2z = kx - jx, ky - jy, kz - jz
        dot = v1x * v2x + v1y * v2y + v1z * v2z
        n1 = v1x * v1x + v1y * v1y + v1z * v1z + 1e-8
        n2 = v2x * v2x + v2y * v2y + v2z * v2z + 1e-8
        ov[pl.ds(i * 16, 16)] = dot * _rsqrt16(n1) * _rsqrt16(n2)
        return carry

    lax.fori_loop(0, EPT // 16, tbody, 0)
    pltpu.sync_copy(ov, ct_h.at[sl_w])

    pltpu.sync_copy(qa_h.at[sl_w], i0)
    pltpu.sync_copy(qb_h.at[sl_w], i1)
    pltpu.sync_copy(qc_h.at[sl_w], i2)
    pltpu.sync_copy(qd_h.at[sl_w], i3)

    def qbody(i, carry):
        a16 = i0[pl.ds(i * 16, 16)]
        b16 = i1[pl.ds(i * 16, 16)]
        c16 = i2[pl.ds(i * 16, 16)]
        d16 = i3[pl.ds(i * 16, 16)]
        ax, ay, az = gxyz(a16)
        bx, by, bz = gxyz(b16)
        cx, cy, cz = gxyz(c16)
        dx, dy, dz = gxyz(d16)
        u1x, u1y, u1z = ax - bx, ay - by, az - bz
        u2x, u2y, u2z = dx - cx, dy - cy, dz - cz
        dot = u1x * u2x + u1y * u2y + u1z * u2z
        n1 = u1x * u1x + u1y * u1y + u1z * u1z + 1e-8
        n2 = u2x * u2x + u2y * u2y + u2z * u2z + 1e-8
        ov[pl.ds(i * 16, 16)] = dot * _rsqrt16(n1) * _rsqrt16(n2)
        return carry

    lax.fori_loop(0, EPT // 16, qbody, 0)
    pltpu.sync_copy(ov, cq_h.at[sl_w])


_geom = pl.kernel(
    _geom_body,
    out_type=[jax.ShapeDtypeStruct((N_EDGES,), jnp.float32)] * 3,
    mesh=_mesh,
    compiler_params=_sc_params,
    scratch_types=(
        [pltpu.VMEM((N_NODES,), jnp.float32)] * 3
        + [pltpu.VMEM((EPT,), jnp.int32)] * 4
        + [pltpu.VMEM((EPT,), jnp.float32)]
    ),
)


def _msg_body(xm_h, xt_h, xq_h, g_h, ct_h, cq_h,
              src_h, dst_h, ti_h, tj_h, tk_h, qa_h, qb_h, qd_h,
              acc_h, accsh, g1i, g2i, wi, scal, ra, rb, gv, zv, sem):
    c = lax.axis_index("c")
    s = lax.axis_index("s")
    w = c * NS + s
    z16 = jnp.zeros((16,), jnp.float32)

    def zrow(r, carry):
        for cc in range(HID // 16):
            zv[r, pl.ds(cc * 16, 16)] = z16
        return carry

    lax.fori_loop(0, 64, zrow, 0)

    def zcopy(k, carry):
        pltpu.sync_copy(zv, accsh.at[pl.ds(s * NPT + k * 64, 64)])
        return carry

    lax.fori_loop(0, NPT // 64, zcopy, 0)
    plsc.subcore_barrier()

    pltpu.sync_copy(src_h.at[w], g1i)
    pltpu.sync_copy(dst_h.at[w], wi)

    def echunk(j, carry):
        pltpu.async_copy(xm_h.at[g1i.at[j]], ra, sem).wait()
        pltpu.sync_copy(g_h.at[pl.ds(w * EPT + j * CH, CH)], gv)

        def erow(r, carry2):
            for cc in range(HID // 16):
                sl = pl.ds(cc * 16, 16)
                ra[r, sl] = ra[r, sl] * gv[r, sl]
            return carry2

        lax.fori_loop(0, CH, erow, 0)
        pltpu.sync_copy(ra, accsh.at[wi.at[j]], add=True)
        return carry

    lax.fori_loop(0, NCH, echunk, 0)

    pltpu.sync_copy(ti_h.at[w], g1i)
    pltpu.sync_copy(tk_h.at[w], g2i)
    pltpu.sync_copy(tj_h.at[w], wi)
    pltpu.sync_copy(ct_h.at[pl.ds(w * EPT, EPT)], scal)

    def tchunk(j, carry):
        pltpu.async_copy(xt_h.at[g1i.at[j]], ra, sem).wait()
        pltpu.async_copy(xt_h.at[g2i.at[j]], rb, sem).wait()

        def trow(r, carry2):
            idx = jnp.zeros((16,), jnp.int32) + (j * CH + r)
            sc = plsc.load_gather(scal, [idx])
            for cc in range(HID // 16):
                sl = pl.ds(cc * 16, 16)
                ra[r, sl] = ra[r, sl] * rb[r, sl] * sc
            return carry2

        lax.fori_loop(0, CH, trow, 0)
        pltpu.sync_copy(ra, accsh.at[wi.at[j]], add=True)
        return carry

    lax.fori_loop(0, NCH, tchunk, 0)

    pltpu.sync_copy(qa_h.at[w], g1i)
    pltpu.sync_copy(qd_h.at[w], g2i)
    pltpu.sync_copy(qb_h.at[w], wi)
    pltpu.sync_copy(cq_h.at[pl.ds(w * EPT, EPT)], scal)

    def qchunk(j, carry):
        pltpu.async_copy(xq_h.at[g1i.at[j]], ra, sem).wait()
        pltpu.async_copy(xq_h.at[g2i.at[j]], rb, sem).wait()

        def qrow(r, carry2):
            idx = jnp.zeros((16,), jnp.int32) + (j * CH + r)
            sc = plsc.load_gather(scal, [idx])
            for cc in range(HID // 16):
                sl = pl.ds(cc * 16, 16)
                ra[r, sl] = ra[r, sl] * rb[r, sl] * sc
            return carry2

        lax.fori_loop(0, CH, qrow, 0)
        pltpu.sync_copy(ra, accsh.at[wi.at[j]], add=True)
        return carry

    lax.fori_loop(0, NCH, qchunk, 0)

    plsc.subcore_barrier()
    pltpu.sync_copy(accsh.at[pl.ds(s * NPT, NPT)],
                    acc_h.at[pl.ds(c * NPAD + s * NPT, NPT)])


_msg = pl.kernel(
    _msg_body,
    out_type=jax.ShapeDtypeStruct((NC * NPAD, HID), jnp.float32),
    mesh=_mesh,
    compiler_params=_sc_params,
    scratch_types=(
        [pltpu.VMEM_SHARED((NPAD, HID), jnp.float32)]
        + [pltpu.VMEM((NCH, CH), jnp.int32)] * 3
        + [pltpu.VMEM((EPT,), jnp.float32)]
        + [pltpu.VMEM((CH, HID), jnp.float32)] * 3
        + [pltpu.VMEM((64, HID), jnp.float32)]
        + [pltpu.SemaphoreType.DMA]
    ),
)


def _proj(x, wcat):
    n, din = x.shape
    blk = 1000

    def body(x_ref, w_ref, o_ref):
        o_ref[...] = jnp.dot(x_ref[...], w_ref[...],
                             preferred_element_type=jnp.float32)

    return pl.pallas_call(
        body,
        grid=(n // blk,),
        in_specs=[pl.BlockSpec((blk, din), lambda i: (i, 0)),
                  pl.BlockSpec((din, 4 * HID), lambda i: (0, 0))],
        out_specs=pl.BlockSpec((blk, 4 * HID), lambda i: (i, 0)),
        out_shape=jax.ShapeDtypeStruct((n, 4 * HID), jnp.float32),
    )(x, wcat)


def _gate(ea, we, wd, dist):
    blk = 4000

    def body(ea_ref, we_ref, wd_ref, d_ref, o_ref):
        v = jnp.dot(ea_ref[...], we_ref[...],
                    preferred_element_type=jnp.float32)
        v = v + d_ref[...] * wd_ref[...]
        o_ref[...] = 1.0 / (1.0 + jnp.exp(-v))

    return pl.pallas_call(
        body,
        grid=(N_EDGES // blk,),
        in_specs=[pl.BlockSpec((blk, 16), lambda i: (i, 0)),
                  pl.BlockSpec((16, HID), lambda i: (0, 0)),
                  pl.BlockSpec((1, HID), lambda i: (0, 0)),
                  pl.BlockSpec((blk, 1), lambda i: (i, 0))],
        out_specs=pl.BlockSpec((blk, HID), lambda i: (i, 0)),
        out_shape=jax.ShapeDtypeStruct((N_EDGES, HID), jnp.float32),
    )(ea, we, wd, dist)


def _combine(xs, a0, a1):
    blk = 1000

    def body(xs_ref, a0_ref, a1_ref, o_ref):
        v = xs_ref[...] + a0_ref[...] + a1_ref[...]
        o_ref[...] = jnp.where(v > 0, v, jnp.exp(jnp.minimum(v, 0.0)) - 1.0)

    return pl.pallas_call(
        body,
        grid=(N_NODES // blk,),
        in_specs=[pl.BlockSpec((blk, HID), lambda i: (i, 0)),
                  pl.BlockSpec((blk, HID), lambda i: (i, 0)),
                  pl.BlockSpec((blk, HID), lambda i: (i, 0))],
        out_specs=pl.BlockSpec((blk, HID), lambda i: (i, 0)),
        out_shape=jax.ShapeDtypeStruct((N_NODES, HID), jnp.float32),
    )(xs, a0, a1)


def _readout(x, batch_r):
    blk = 1000
    ng = N_NODES // blk

    def body(x_ref, b_ref, s_ref, c_ref):
        i = pl.program_id(0)
        seg = lax.broadcasted_iota(jnp.int32, (NB, blk), 0)
        oh = (b_ref[0] == seg).astype(jnp.float32)
        ps = jnp.dot(oh, x_ref[...], preferred_element_type=jnp.float32)
        pc = jnp.sum(oh, axis=1, keepdims=True)

        @pl.when(i == 0)
        def _():
            s_ref[...] = ps
            c_ref[...] = pc

        @pl.when(i > 0)
        def _():
            s_ref[...] += ps
            c_ref[...] += pc

    return pl.pallas_call(
        body,
        grid=(ng,),
        in_specs=[pl.BlockSpec((blk, HID), lambda i: (i, 0)),
                  pl.BlockSpec((1, 1, blk), lambda i: (i, 0, 0))],
        out_specs=[pl.BlockSpec((NB, HID), lambda i: (0, 0)),
                   pl.BlockSpec((NB, 1), lambda i: (0, 0))],
        out_shape=[jax.ShapeDtypeStruct((NB, HID), jnp.float32),
                   jax.ShapeDtypeStruct((NB, 1), jnp.float32)],
    )(x, batch_r)


def _head_body(s1_ref, c1_ref, s2_ref, c2_ref, mr_ref, tp_ref, tok_ref,
               wq_ref, wk_ref, wv_ref, wo_ref,
               l1g_ref, l1b_ref, l2g_ref, l2b_ref,
               w1_ref, b1_ref, w2_ref, b2_ref,
               bw_ref, bwm_ref, bb_ref,
               mw1_ref, mb1_ref, mw2_ref, mb2_ref, o_ref):
    f32 = jnp.float32

    def ln(v, g, b):
        m = jnp.mean(v, axis=-1, keepdims=True)
        var = jnp.mean((v - m) * (v - m), axis=-1, keepdims=True)
        return (v - m) / jnp.sqrt(var + 1e-5) * g + b

    def mkr(s_ref, c_ref):
        sv = s_ref[...]
        mean = sv / jnp.maximum(c_ref[...], 1.0)
        return jnp.concatenate([mean, sv], axis=1)

    h0 = jnp.broadcast_to(tok_ref[...], (NB, ENC_DIM))
    h1 = mkr(s1_ref, c1_ref)
    h2 = mkr(s2_ref, c2_ref)

    hd = ENC_DIM // ENC_HEADS
    hm = (lax.broadcasted_iota(jnp.int32, (ENC_DIM, ENC_HEADS), 0) // hd
          == lax.broadcasted_iota(jnp.int32, (ENC_DIM, ENC_HEADS), 1)
          ).astype(f32)
    hmT = (lax.broadcasted_iota(jnp.int32, (ENC_HEADS, ENC_DIM), 0)
           == lax.broadcasted_iota(jnp.int32, (ENC_HEADS, ENC_DIM), 1) // hd
           ).astype(f32)
    scale = 1.0 / (hd ** 0.5)

    hs = [h0, h1, h2]
    for l in range(ENC_LAYERS):
        wq, wk, wv, wo = wq_ref[l], wk_ref[l], wv_ref[l], wo_ref[l]
        qs = [jnp.dot(h, wq, preferred_element_type=f32) for h in hs]
        ks = [jnp.dot(h, wk, preferred_element_type=f32) for h in hs]
        vs = [jnp.dot(h, wv, preferred_element_type=f32) for h in hs]
        sc = [[jnp.dot(qs[i] * ks[j], hm, preferred_element_type=f32) * scale
               for j in range(3)] for i in range(3)]
        new_hs = []
        for i in range(3):
            m = jnp.maximum(jnp.maximum(sc[i][0], sc[i][1]), sc[i][2])
            es = [jnp.exp(sc[i][j] - m) for j in range(3)]
            den = es[0] + es[1] + es[2]
            o = jnp.zeros((NB, ENC_DIM), f32)
            for j in range(3):
                att = es[j] / den
                o = o + jnp.dot(att, hmT, preferred_element_type=f32) * vs[j]
            o = jnp.dot(o, wo, preferred_element_type=f32)
            h = ln(hs[i] + o, l1g_ref[l], l1b_ref[l])
            f = jnp.dot(h, w1_ref[l], preferred_element_type=f32) + b1_ref[l]
            f = jnp.maximum(f, 0.0)
            f = jnp.dot(f, w2_ref[l], preferred_element_type=f32) + b2_ref[l]
            new_hs.append(ln(h + f, l2g_ref[l], l2b_ref[l]))
        hs = new_hs

    inter = hs[0]
    mr = mr_ref[...]
    en = (jnp.dot(inter, bw_ref[...], preferred_element_type=f32)
          + mr * bwm_ref[0:1, :] + (1.0 - mr) * bwm_ref[1:2, :]
          + bb_ref[...])
    logits = -en / (tp_ref[...] + 1.0)
    m = jnp.max(logits, axis=-1, keepdims=True)
    e = jnp.exp(logits - m)
    dist = e / jnp.sum(e, axis=-1, keepdims=True)
    hmv = jnp.dot(dist, mw1_ref[...], preferred_element_type=f32) + mb1_ref[...]
    hmv = jnp.where(hmv > 0, hmv, jnp.exp(jnp.minimum(hmv, 0.0)) - 1.0)
    o_ref[...] = (jnp.dot(hmv, mw2_ref[...], preferred_element_type=f32)
                  + mb2_ref[...])


def _head(s1, c1, s2, c2, mr, tp, p):
    enc = p['enc']
    bwm = jnp.concatenate(
        [p['boltz']['W'][2 * HID:2 * HID + 1],
         p['boltz']['W'][2 * HID + 1:2 * HID + 2],
         jnp.zeros((6, N_ENERGY), jnp.float32)], axis=0)
    args = (
        s1, c1, s2, c2, mr.reshape(NB, 1), tp.reshape(NB, 1),
        p['repr_token'],
        enc['Wq'], enc['Wk'], enc['Wv'], enc['Wo'],
        enc['ln1_g'].reshape(ENC_LAYERS, 1, ENC_DIM),
        enc['ln1_b'].reshape(ENC_LAYERS, 1, ENC_DIM),
        enc['ln2_g'].reshape(ENC_LAYERS, 1, ENC_DIM),
        enc['ln2_b'].reshape(ENC_LAYERS, 1, ENC_DIM),
        enc['W1'], enc['b1'].reshape(ENC_LAYERS, 1, FF),
        enc['W2'], enc['b2'].reshape(ENC_LAYERS, 1, ENC_DIM),
        p['boltz']['W'][:2 * HID], bwm,
        p['boltz']['b'].reshape(1, N_ENERGY),
        p['mlp']['W1'], p['mlp']['b1'].reshape(1, 128),
        p['mlp']['W2'], p['mlp']['b2'].reshape(1, 1),
    )
    return pl.pallas_call(
        _head_body,
        out_shape=jax.ShapeDtypeStruct((NB, 1), jnp.float32),
    )(*args)


def _graph_repr(params, x, pos, ea, ei, ti, qi, batch):
    i32 = jnp.int32
    se = ei[0].astype(i32)
    de = ei[1].astype(i32)
    t0 = ti[0].astype(i32)
    t1 = ti[1].astype(i32)
    t2 = ti[2].astype(i32)
    q0 = qi[0].astype(i32)
    q1 = qi[1].astype(i32)
    q2 = qi[2].astype(i32)
    q3 = qi[3].astype(i32)

    px, py, pz = pos[:, 0], pos[:, 1], pos[:, 2]
    dist, ct, cq = _geom(px, py, pz, se, de, t0, t1, t2, q0, q1, q2, q3)

    dist2d = dist.reshape(N_EDGES, 1)
    src3 = se.reshape(NW, NCH, CH)
    dst3 = de.reshape(NW, NCH, CH)
    ti3 = t0.reshape(NW, NCH, CH)
    tj3 = t1.reshape(NW, NCH, CH)
    tk3 = t2.reshape(NW, NCH, CH)
    qa3 = q0.reshape(NW, NCH, CH)
    qb3 = q1.reshape(NW, NCH, CH)
    qd3 = q3.reshape(NW, NCH, CH)

    h = x
    for p in params['gnn']:
        wcat = jnp.concatenate(
            [p['W_self'], p['W_msg'], p['W_t'], p['W_q']], axis=1)
        pr = _proj(h, wcat)
        xs = pr[:, :HID]
        xm = pr[:, HID:2 * HID]
        xt = pr[:, 2 * HID:3 * HID]
        xq = pr[:, 3 * HID:]
        g = _gate(ea, p['W_e'], p['W_d'], dist2d)
        acc = _msg(xm, xt, xq, g, ct, cq,
                   src3, dst3, ti3, tj3, tk3, qa3, qb3, qd3)
        h = _combine(xs, acc[:N_NODES], acc[NPAD:NPAD + N_NODES])

    batch_r = batch.astype(i32).reshape(N_NODES // 1000, 1, 1000)
    s, cnt = _readout(h, batch_r)
    return s, cnt


def kernel(x1, pos1, edge_attr1, edge_index1, triple_index1, quadra_index1,
           batch1, x2, pos2, edge_attr2, edge_index2, triple_index2,
           quadra_index2, batch2, molar_ratio, temps, params):
    s1, c1 = _graph_repr(params, x1, pos1, edge_attr1, edge_index1,
                         triple_index1, quadra_index1, batch1)
    s2, c2 = _graph_repr(params, x2, pos2, edge_attr2, edge_index2,
                         triple_index2, quadra_index2, batch2)
    out = _head(s1, c1, s2, c2, molar_ratio, temps, params)
    return out[:, 0]


# speedup vs baseline: 9.5267x; 1.6872x over previous
import functools

import jax
import jax.numpy as jnp
from jax import lax
from jax.experimental import pallas as pl
from jax.experimental.pallas import tpu as pltpu
from jax.experimental.pallas import tpu_sc as plsc

N_NODES = 10000
N_EDGES = 320000
NB = 256
HID = 64
ENC_DIM = 128
ENC_HEADS = 8
ENC_LAYERS = 3
FF = 512
N_ENERGY = 64

NC = 2
NS = 16
NW = NC * NS
EPT = N_EDGES // NW
CH = 80
NCH = EPT // CH
NPAD = 10240
NPT = NPAD // NS

_mesh = plsc.VectorSubcoreMesh(core_axis_name="c", subcore_axis_name="s")
_sc_params = pltpu.CompilerParams(needs_layout_passes=False, use_tc_tiling_on_sc=False)


def _rsqrt16(x):
    i = plsc.bitcast(x, jnp.int32)
    i = jnp.int32(0x5F3759DF) - (i >> 1)
    y = plsc.bitcast(i, jnp.float32)
    y = y * (1.5 - 0.5 * x * y * y)
    y = y * (1.5 - 0.5 * x * y * y)
    y = y * (1.5 - 0.5 * x * y * y)
    return y


def _geom_body(px_h, py_h, pz_h, se_h, de_h, ti_h, tj_h, tk_h, qa_h, qb_h,
               qc_h, qd_h, dist_h, ct_h, cq_h, px, py, pz, i0, i1, i2, i3,
               ov):
    c = lax.axis_index("c")
    s = lax.axis_index("s")
    w = c * NS + s
    sl_w = pl.ds(w * EPT, EPT)
    pltpu.sync_copy(px_h, px)
    pltpu.sync_copy(py_h, py)
    pltpu.sync_copy(pz_h, pz)

    def gxyz(idx):
        return (plsc.load_gather(px, [idx]),
                plsc.load_gather(py, [idx]),
                plsc.load_gather(pz, [idx]))

    pltpu.sync_copy(se_h.at[sl_w], i0)
    pltpu.sync_copy(de_h.at[sl_w], i1)

    @plsc.parallel_loop(0, EPT // 16, 1, unroll=4)
    def ebody(i):
        a16 = i0[pl.ds(i * 16, 16)]
        b16 = i1[pl.ds(i * 16, 16)]
        ax, ay, az = gxyz(a16)
        bx, by, bz = gxyz(b16)
        ux, uy, uz = ax - bx, ay - by, az - bz
        n2 = ux * ux + uy * uy + uz * uz + 1e-12
        ov[pl.ds(i * 16, 16)] = n2 * _rsqrt16(n2)

    pltpu.sync_copy(ov, dist_h.at[sl_w])

    pltpu.sync_copy(ti_h.at[sl_w], i0)
    pltpu.sync_copy(tj_h.at[sl_w], i1)
    pltpu.sync_copy(tk_h.at[sl_w], i2)

    @plsc.parallel_loop(0, EPT // 16, 1, unroll=4)
    def tbody(i):
        a16 = i0[pl.ds(i * 16, 16)]
        b16 = i1[pl.ds(i * 16, 16)]
        c16 = i2[pl.ds(i * 16, 16)]
        ix, iy, iz = gxyz(a16)
        jx, jy, jz = gxyz(b16)
        kx, ky, kz = gxyz(c16)
        v1x, v1y, v1z = ix - jx, iy - jy, iz - jz
        v2x, v2y, v2z = kx - jx, ky - jy, kz - jz
        dot = v1x * v2x + v1y * v2y + v1z * v2z
        n1 = v1x * v1x + v1y * v1y + v1z * v1z + 1e-8
        n2 = v2x * v2x + v2y * v2y + v2z * v2z + 1e-8
        ov[pl.ds(i * 16, 16)] = dot * _rsqrt16(n1) * _rsqrt16(n2)

    pltpu.sync_copy(ov, ct_h.at[sl_w])

    pltpu.sync_copy(qa_h.at[sl_w], i0)
    pltpu.sync_copy(qb_h.at[sl_w], i1)
    pltpu.sync_copy(qc_h.at[sl_w], i2)
    pltpu.sync_copy(qd_h.at[sl_w], i3)

    @plsc.parallel_loop(0, EPT // 16, 1, unroll=4)
    def qbody(i):
        a16 = i0[pl.ds(i * 16, 16)]
        b16 = i1[pl.ds(i * 16, 16)]
        c16 = i2[pl.ds(i * 16, 16)]
        d16 = i3[pl.ds(i * 16, 16)]
        ax, ay, az = gxyz(a16)
        bx, by, bz = gxyz(b16)
        cx, cy, cz = gxyz(c16)
        dx, dy, dz = gxyz(d16)
        u1x, u1y, u1z = ax - bx, ay - by, az - bz
        u2x, u2y, u2z = dx - cx, dy - cy, dz - cz
        dot = u1x * u2x + u1y * u2y + u1z * u2z
        n1 = u1x * u1x + u1y * u1y + u1z * u1z + 1e-8
        n2 = u2x * u2x + u2y * u2y + u2z * u2z + 1e-8
        ov[pl.ds(i * 16, 16)] = dot * _rsqrt16(n1) * _rsqrt16(n2)

    pltpu.sync_copy(ov, cq_h.at[sl_w])


_geom = pl.kernel(
    _geom_body,
    out_type=[jax.ShapeDtypeStruct((N_EDGES,), jnp.float32)] * 3,
    mesh=_mesh,
    compiler_params=_sc_params,
    scratch_types=(
        [pltpu.VMEM((N_NODES,), jnp.float32)] * 3
        + [pltpu.VMEM((EPT,), jnp.int32)] * 4
        + [pltpu.VMEM((EPT,), jnp.float32)]
    ),
)


def _msg_body(xm_h, xt_h, xq_h, g_h, ct_h, cq_h,
              src_h, dst_h, ti_h, tj_h, tk_h, qa_h, qb_h, qd_h,
              acc_h, accsh, g1i, g2i, wi, scal,
              ra0, ra1, rb0, rb1, mb0, mb1, zv, gs0, gs1):
    c = lax.axis_index("c")
    s = lax.axis_index("s")
    w = c * NS + s
    z16 = jnp.zeros((16,), jnp.float32)
    ra = (ra0, ra1)
    rb = (rb0, rb1)
    mb = (mb0, mb1)
    gsem = (gs0, gs1)

    def zrow(r, carry):
        for cc in range(HID // 16):
            zv[r, pl.ds(cc * 16, 16)] = z16
        return carry

    lax.fori_loop(0, 64, zrow, 0)

    def zcopy(k, carry):
        pltpu.sync_copy(zv, accsh.at[pl.ds(s * NPT + k * 64, 64)])
        return carry

    lax.fori_loop(0, NPT // 64, zcopy, 0)
    plsc.subcore_barrier()

    def phase(tab1, tab2, use_gate):
        def g_start(j, b):
            pltpu.async_copy(tab1.at[g1i.at[j]], ra[b], gsem[b])
            if tab2 is not None:
                pltpu.async_copy(tab2.at[g2i.at[j]], rb[b], gsem[b])

        def g_wait(b):
            pltpu.make_async_copy(tab1.at[pl.ds(0, CH)], ra[b],
                                  gsem[b]).wait()
            if tab2 is not None:
                pltpu.make_async_copy(tab1.at[pl.ds(0, CH)], rb[b],
                                      gsem[b]).wait()

        def proc(j, b):
            if use_gate:
                pltpu.sync_copy(g_h.at[pl.ds(w * EPT + j * CH, CH)], mb[b])

                def erow(r, carry2):
                    for cc in range(HID // 16):
                        sl = pl.ds(cc * 16, 16)
                        mb[b][r, sl] = mb[b][r, sl] * ra[b][r, sl]
                    return carry2

                lax.fori_loop(0, CH, erow, 0)
            else:
                def trow(r, carry2):
                    idx = jnp.zeros((16,), jnp.int32) + (j * CH + r)
                    sc = plsc.load_gather(scal, [idx])
                    for cc in range(HID // 16):
                        sl = pl.ds(cc * 16, 16)
                        mb[b][r, sl] = ra[b][r, sl] * rb[b][r, sl] * sc
                    return carry2

                lax.fori_loop(0, CH, trow, 0)
            pltpu.sync_copy(mb[b], accsh.at[wi.at[j]], add=True)

        g_start(0, 0)

        def pair(t, carry):
            j0 = 2 * t
            g_start(j0 + 1, 1)
            g_wait(0)
            proc(j0, 0)
            g_start(j0 + 2, 0)
            g_wait(1)
            proc(j0 + 1, 1)
            return carry

        lax.fori_loop(0, (NCH - 1) // 2, pair, 0)
        g_wait(0)
        proc(NCH - 1, 0)

    pltpu.sync_copy(src_h.at[w], g1i)
    pltpu.sync_copy(dst_h.at[w], wi)
    phase(xm_h, None, True)

    pltpu.sync_copy(ti_h.at[w], g1i)
    pltpu.sync_copy(tk_h.at[w], g2i)
    pltpu.sync_copy(tj_h.at[w], wi)
    pltpu.sync_copy(ct_h.at[pl.ds(w * EPT, EPT)], scal)
    phase(xt_h, xt_h, False)

    pltpu.sync_copy(qa_h.at[w], g1i)
    pltpu.sync_copy(qd_h.at[w], g2i)
    pltpu.sync_copy(qb_h.at[w], wi)
    pltpu.sync_copy(cq_h.at[pl.ds(w * EPT, EPT)], scal)
    phase(xq_h, xq_h, False)

    plsc.subcore_barrier()
    pltpu.sync_copy(accsh.at[pl.ds(s * NPT, NPT)],
                    acc_h.at[pl.ds(c * NPAD + s * NPT, NPT)])


_msg = pl.kernel(
    _msg_body,
    out_type=jax.ShapeDtypeStruct((NC * NPAD, HID), jnp.float32),
    mesh=_mesh,
    compiler_params=_sc_params,
    scratch_types=(
        [pltpu.VMEM_SHARED((NPAD, HID), jnp.float32)]
        + [pltpu.VMEM((NCH, CH), jnp.int32)] * 3
        + [pltpu.VMEM((EPT,), jnp.float32)]
        + [pltpu.VMEM((CH, HID), jnp.float32)] * 6
        + [pltpu.VMEM((64, HID), jnp.float32)]
        + [pltpu.SemaphoreType.DMA] * 2
    ),
)


def _proj(x, wcat):
    n, din = x.shape
    blk = 1000

    def body(x_ref, w_ref, o_ref):
        o_ref[...] = jnp.dot(x_ref[...], w_ref[...],
                             preferred_element_type=jnp.float32)

    return pl.pallas_call(
        body,
        grid=(n // blk,),
        in_specs=[pl.BlockSpec((blk, din), lambda i: (i, 0)),
                  pl.BlockSpec((din, 4 * HID), lambda i: (0, 0))],
        out_specs=pl.BlockSpec((blk, 4 * HID), lambda i: (i, 0)),
        out_shape=jax.ShapeDtypeStruct((n, 4 * HID), jnp.float32),
    )(x, wcat)


def _gate(ea, we, wd, dist):
    blk = 4000

    def body(ea_ref, we_ref, wd_ref, d_ref, o_ref):
        v = jnp.dot(ea_ref[...], we_ref[...],
                    preferred_element_type=jnp.float32)
        v = v + d_ref[...] * wd_ref[...]
        o_ref[...] = 1.0 / (1.0 + jnp.exp(-v))

    return pl.pallas_call(
        body,
        grid=(N_EDGES // blk,),
        in_specs=[pl.BlockSpec((blk, 16), lambda i: (i, 0)),
                  pl.BlockSpec((16, HID), lambda i: (0, 0)),
                  pl.BlockSpec((1, HID), lambda i: (0, 0)),
                  pl.BlockSpec((blk, 1), lambda i: (i, 0))],
        out_specs=pl.BlockSpec((blk, HID), lambda i: (i, 0)),
        out_shape=jax.ShapeDtypeStruct((N_EDGES, HID), jnp.float32),
    )(ea, we, wd, dist)


def _combine(xs, a0, a1):
    blk = 1000

    def body(xs_ref, a0_ref, a1_ref, o_ref):
        v = xs_ref[...] + a0_ref[...] + a1_ref[...]
        o_ref[...] = jnp.where(v > 0, v, jnp.exp(jnp.minimum(v, 0.0)) - 1.0)

    return pl.pallas_call(
        body,
        grid=(N_NODES // blk,),
        in_specs=[pl.BlockSpec((blk, HID), lambda i: (i, 0)),
                  pl.BlockSpec((blk, HID), lambda i: (i, 0)),
                  pl.BlockSpec((blk, HID), lambda i: (i, 0))],
        out_specs=pl.BlockSpec((blk, HID), lambda i: (i, 0)),
        out_shape=jax.ShapeDtypeStruct((N_NODES, HID), jnp.float32),
    )(xs, a0, a1)


def _readout(x, batch_r):
    blk = 1000
    ng = N_NODES // blk

    def body(x_ref, b_ref, s_ref, c_ref):
        i = pl.program_id(0)
        seg = lax.broadcasted_iota(jnp.int32, (NB, blk), 0)
        oh = (b_ref[0] == seg).astype(jnp.float32)
        ps = jnp.dot(oh, x_ref[...], preferred_element_type=jnp.float32)
        pc = jnp.sum(oh, axis=1, keepdims=True)

        @pl.when(i == 0)
        def _():
            s_ref[...] = ps
            c_ref[...] = pc

        @pl.when(i > 0)
        def _():
            s_ref[...] += ps
            c_ref[...] += pc

    return pl.pallas_call(
        body,
        grid=(ng,),
        in_specs=[pl.BlockSpec((blk, HID), lambda i: (i, 0)),
                  pl.BlockSpec((1, 1, blk), lambda i: (i, 0, 0))],
        out_specs=[pl.BlockSpec((NB, HID), lambda i: (0, 0)),
                   pl.BlockSpec((NB, 1), lambda i: (0, 0))],
        out_shape=[jax.ShapeDtypeStruct((NB, HID), jnp.float32),
                   jax.ShapeDtypeStruct((NB, 1), jnp.float32)],
    )(x, batch_r)


def _head_body(s1_ref, c1_ref, s2_ref, c2_ref, mr_ref, tp_ref, tok_ref,
               wq_ref, wk_ref, wv_ref, wo_ref,
               l1g_ref, l1b_ref, l2g_ref, l2b_ref,
               w1_ref, b1_ref, w2_ref, b2_ref,
               bw_ref, bwm_ref, bb_ref,
               mw1_ref, mb1_ref, mw2_ref, mb2_ref, o_ref):
    f32 = jnp.float32

    def ln(v, g, b):
        m = jnp.mean(v, axis=-1, keepdims=True)
        var = jnp.mean((v - m) * (v - m), axis=-1, keepdims=True)
        return (v - m) / jnp.sqrt(var + 1e-5) * g + b

    def mkr(s_ref, c_ref):
        sv = s_ref[...]
        mean = sv / jnp.maximum(c_ref[...], 1.0)
        return jnp.concatenate([mean, sv], axis=1)

    h0 = jnp.broadcast_to(tok_ref[...], (NB, ENC_DIM))
    h1 = mkr(s1_ref, c1_ref)
    h2 = mkr(s2_ref, c2_ref)

    hd = ENC_DIM // ENC_HEADS
    hm = (lax.broadcasted_iota(jnp.int32, (ENC_DIM, ENC_HEADS), 0) // hd
          == lax.broadcasted_iota(jnp.int32, (ENC_DIM, ENC_HEADS), 1)
          ).astype(f32)
    hmT = (lax.broadcasted_iota(jnp.int32, (ENC_HEADS, ENC_DIM), 0)
           == lax.broadcasted_iota(jnp.int32, (ENC_HEADS, ENC_DIM), 1) // hd
           ).astype(f32)
    scale = 1.0 / (hd ** 0.5)

    hs = [h0, h1, h2]
    for l in range(ENC_LAYERS):
        wq, wk, wv, wo = wq_ref[l], wk_ref[l], wv_ref[l], wo_ref[l]
        qs = [jnp.dot(h, wq, preferred_element_type=f32) for h in hs]
        ks = [jnp.dot(h, wk, preferred_element_type=f32) for h in hs]
        vs = [jnp.dot(h, wv, preferred_element_type=f32) for h in hs]
        sc = [[jnp.dot(qs[i] * ks[j], hm, preferred_element_type=f32) * scale
               for j in range(3)] for i in range(3)]
        new_hs = []
        for i in range(3):
            m = jnp.maximum(jnp.maximum(sc[i][0], sc[i][1]), sc[i][2])
            es = [jnp.exp(sc[i][j] - m) for j in range(3)]
            den = es[0] + es[1] + es[2]
            o = jnp.zeros((NB, ENC_DIM), f32)
            for j in range(3):
                att = es[j] / den
                o = o + jnp.dot(att, hmT, preferred_element_type=f32) * vs[j]
            o = jnp.dot(o, wo, preferred_element_type=f32)
            h = ln(hs[i] + o, l1g_ref[l], l1b_ref[l])
            f = jnp.dot(h, w1_ref[l], preferred_element_type=f32) + b1_ref[l]
            f = jnp.maximum(f, 0.0)
            f = jnp.dot(f, w2_ref[l], preferred_element_type=f32) + b2_ref[l]
            new_hs.append(ln(h + f, l2g_ref[l], l2b_ref[l]))
        hs = new_hs

    inter = hs[0]
    mr = mr_ref[...]
    en = (jnp.dot(inter, bw_ref[...], preferred_element_type=f32)
          + mr * bwm_ref[0:1, :] + (1.0 - mr) * bwm_ref[1:2, :]
          + bb_ref[...])
    logits = -en / (tp_ref[...] + 1.0)
    m = jnp.max(logits, axis=-1, keepdims=True)
    e = jnp.exp(logits - m)
    dist = e / jnp.sum(e, axis=-1, keepdims=True)
    hmv = jnp.dot(dist, mw1_ref[...], preferred_element_type=f32) + mb1_ref[...]
    hmv = jnp.where(hmv > 0, hmv, jnp.exp(jnp.minimum(hmv, 0.0)) - 1.0)
    o_ref[...] = (jnp.dot(hmv, mw2_ref[...], preferred_element_type=f32)
                  + mb2_ref[...])


def _head(s1, c1, s2, c2, mr, tp, p):
    enc = p['enc']
    bwm = jnp.concatenate(
        [p['boltz']['W'][2 * HID:2 * HID + 1],
         p['boltz']['W'][2 * HID + 1:2 * HID + 2],
         jnp.zeros((6, N_ENERGY), jnp.float32)], axis=0)
    args = (
        s1, c1, s2, c2, mr.reshape(NB, 1), tp.reshape(NB, 1),
        p['repr_token'],
        enc['Wq'], enc['Wk'], enc['Wv'], enc['Wo'],
        enc['ln1_g'].reshape(ENC_LAYERS, 1, ENC_DIM),
        enc['ln1_b'].reshape(ENC_LAYERS, 1, ENC_DIM),
        enc['ln2_g'].reshape(ENC_LAYERS, 1, ENC_DIM),
        enc['ln2_b'].reshape(ENC_LAYERS, 1, ENC_DIM),
        enc['W1'], enc['b1'].reshape(ENC_LAYERS, 1, FF),
        enc['W2'], enc['b2'].reshape(ENC_LAYERS, 1, ENC_DIM),
        p['boltz']['W'][:2 * HID], bwm,
        p['boltz']['b'].reshape(1, N_ENERGY),
        p['mlp']['W1'], p['mlp']['b1'].reshape(1, 128),
        p['mlp']['W2'], p['mlp']['b2'].reshape(1, 1),
    )
    return pl.pallas_call(
        _head_body,
        out_shape=jax.ShapeDtypeStruct((NB, 1), jnp.float32),
    )(*args)


def _graph_repr(params, x, pos, ea, ei, ti, qi, batch):
    i32 = jnp.int32
    se = ei[0].astype(i32)
    de = ei[1].astype(i32)
    t0 = ti[0].astype(i32)
    t1 = ti[1].astype(i32)
    t2 = ti[2].astype(i32)
    q0 = qi[0].astype(i32)
    q1 = qi[1].astype(i32)
    q2 = qi[2].astype(i32)
    q3 = qi[3].astype(i32)

    px, py, pz = pos[:, 0], pos[:, 1], pos[:, 2]
    dist, ct, cq = _geom(px, py, pz, se, de, t0, t1, t2, q0, q1, q2, q3)

    dist2d = dist.reshape(N_EDGES, 1)
    src3 = se.reshape(NW, NCH, CH)
    dst3 = de.reshape(NW, NCH, CH)
    ti3 = t0.reshape(NW, NCH, CH)
    tj3 = t1.reshape(NW, NCH, CH)
    tk3 = t2.reshape(NW, NCH, CH)
    qa3 = q0.reshape(NW, NCH, CH)
    qb3 = q1.reshape(NW, NCH, CH)
    qd3 = q3.reshape(NW, NCH, CH)

    h = x
    for p in params['gnn']:
        wcat = jnp.concatenate(
            [p['W_self'], p['W_msg'], p['W_t'], p['W_q']], axis=1)
        pr = _proj(h, wcat)
        xs = pr[:, :HID]
        xm = pr[:, HID:2 * HID]
        xt = pr[:, 2 * HID:3 * HID]
        xq = pr[:, 3 * HID:]
        g = _gate(ea, p['W_e'], p['W_d'], dist2d)
        acc = _msg(xm, xt, xq, g, ct, cq,
                   src3, dst3, ti3, tj3, tk3, qa3, qb3, qd3)
        h = _combine(xs, acc[:N_NODES], acc[NPAD:NPAD + N_NODES])

    batch_r = batch.astype(i32).reshape(N_NODES // 1000, 1, 1000)
    s, cnt = _readout(h, batch_r)
    return s, cnt


def kernel(x1, pos1, edge_attr1, edge_index1, triple_index1, quadra_index1,
           batch1, x2, pos2, edge_attr2, edge_index2, triple_index2,
           quadra_index2, batch2, molar_ratio, temps, params):
    s1, c1 = _graph_repr(params, x1, pos1, edge_attr1, edge_index1,
                         triple_index1, quadra_index1, batch1)
    s2, c2 = _graph_repr(params, x2, pos2, edge_attr2, edge_index2,
                         triple_index2, quadra_index2, batch2)
    out = _head(s1, c1, s2, c2, molar_ratio, temps, params)
    return out[:, 0]
